# baseline (device time: 253283 ns/iter reference)
import jax
import jax.numpy as jnp
from jax import lax
from jax.experimental import pallas as pl
from jax.experimental.pallas import tpu as pltpu

T = 1024
D = 1024
F = 2048
E = 4
E_LOCAL = 2
CAP = 320


def _pallas_moe(xg, w1b, w2b):
    def body(xg_ref, w1_ref, w2_ref, out_ref, xpeer, ysend,
             send_sems, recv_sems):
        my_x = lax.axis_index("x")
        my_y = lax.axis_index("y")
        my_z = lax.axis_index("z")
        q = 1 - my_x
        peer = (q, my_y, my_z)

        barrier = pltpu.get_barrier_semaphore()
        pl.semaphore_signal(barrier, inc=1, device_id=peer,
                            device_id_type=pl.DeviceIdType.MESH)
        pl.semaphore_wait(barrier, 1)

        xr = []
        for j in range(E_LOCAL):
            r = pltpu.make_async_remote_copy(
                src_ref=xg_ref.at[pl.ds((q * E_LOCAL + j) * CAP, CAP), :],
                dst_ref=xpeer.at[j],
                send_sem=send_sems.at[j], recv_sem=recv_sems.at[j],
                device_id=peer, device_id_type=pl.DeviceIdType.MESH)
            r.start()
            xr.append(r)

        def ffn(xt, le):
            h = jnp.dot(xt, w1_ref[le], preferred_element_type=jnp.float32)
            h = jnp.maximum(h, 0.0).astype(jnp.bfloat16)
            y = jnp.dot(h, w2_ref[le], preferred_element_type=jnp.float32)
            return y.astype(jnp.bfloat16)

        for le in range(E_LOCAL):
            rows = pl.ds((my_x * E_LOCAL + le) * CAP, CAP)
            out_ref[rows, :] = ffn(xg_ref[rows, :], le)

        yr = []
        for le in range(E_LOCAL):
            xr[le].wait()
            ysend[le] = ffn(xpeer[le], le)
            r = pltpu.make_async_remote_copy(
                src_ref=ysend.at[le],
                dst_ref=out_ref.at[pl.ds((my_x * E_LOCAL + le) * CAP, CAP), :],
                send_sem=send_sems.at[E_LOCAL + le],
                recv_sem=recv_sems.at[E_LOCAL + le],
                device_id=peer, device_id_type=pl.DeviceIdType.MESH)
            r.start()
            yr.append(r)
        for r in yr:
            r.wait()

    return pl.pallas_call(
        body,
        out_shape=jax.ShapeDtypeStruct((E * CAP, D), jnp.bfloat16),
        in_specs=[pl.BlockSpec(memory_space=pltpu.VMEM)] * 3,
        out_specs=pl.BlockSpec(memory_space=pltpu.VMEM),
        scratch_shapes=[
            pltpu.VMEM((E_LOCAL, CAP, D), jnp.bfloat16),
            pltpu.VMEM((E_LOCAL, CAP, D), jnp.bfloat16),
            pltpu.SemaphoreType.DMA((2 * E_LOCAL,)),
            pltpu.SemaphoreType.DMA((2 * E_LOCAL,)),
        ],
        compiler_params=pltpu.CompilerParams(collective_id=0),
    )(xg, w1b, w2b)


def kernel(x, assign, W1, W2):
    xb = x.astype(jnp.bfloat16)
    w1b = W1.astype(jnp.bfloat16)
    w2b = W2.astype(jnp.bfloat16)

    idx = jnp.concatenate(
        [jnp.where(assign == e, size=CAP, fill_value=T)[0] for e in range(E)]
    )
    xg = xb.at[idx, :].get(mode="fill", fill_value=0)
    yg = _pallas_moe(xg, w1b, w2b)
    out = jnp.zeros((T, D), jnp.float32)
    return out.at[idx, :].set(yg.astype(jnp.float32), mode="drop")


# device time: 63415 ns/iter; 3.9941x vs baseline; 3.9941x over previous
import jax
import jax.numpy as jnp
from jax import lax
from jax.experimental import pallas as pl
from jax.experimental.pallas import tpu as pltpu

T = 1024
D = 1024
F = 2048
E = 4
E_LOCAL = 2
CAP = 320
B = E * CAP


def _pallas_moe(xb, dest_row, dest_col, w1b, w2b):
    def body(x_ref, drow_ref, dcol_ref, w1_ref, w2_ref, out_ref,
             pmat, ptmat, xsend, xpeer, ybuckets, ysend,
             send_sems, recv_sems):
        my_x = lax.axis_index("x")
        my_y = lax.axis_index("y")
        my_z = lax.axis_index("z")
        q = 1 - my_x
        peer = (q, my_y, my_z)

        rows = lax.broadcasted_iota(jnp.int32, (B, T), 0)
        pmat[...] = (rows == drow_ref[...]).astype(jnp.bfloat16)
        cols = lax.broadcasted_iota(jnp.int32, (T, B), 1)
        ptmat[...] = (cols == dcol_ref[...]).astype(jnp.bfloat16)
        ybuckets[...] = jnp.zeros((B, D), jnp.bfloat16)

        barrier = pltpu.get_barrier_semaphore()
        pl.semaphore_signal(barrier, inc=1, device_id=peer,
                            device_id_type=pl.DeviceIdType.MESH)
        pl.semaphore_wait(barrier, 1)

        xr = []
        for j in range(E_LOCAL):
            prows = pmat[pl.ds((q * E_LOCAL + j) * CAP, CAP), :]
            xsend[j] = jnp.dot(prows, x_ref[...],
                               preferred_element_type=jnp.float32
                               ).astype(jnp.bfloat16)
            r = pltpu.make_async_remote_copy(
                src_ref=xsend.at[j], dst_ref=xpeer.at[j],
                send_sem=send_sems.at[j], recv_sem=recv_sems.at[j],
                device_id=peer, device_id_type=pl.DeviceIdType.MESH)
            r.start()
            xr.append(r)

        def ffn(xt, le):
            h = jnp.dot(xt, w1_ref[le], preferred_element_type=jnp.float32)
            h = jnp.maximum(h, 0.0).astype(jnp.bfloat16)
            y = jnp.dot(h, w2_ref[le], preferred_element_type=jnp.float32)
            return y.astype(jnp.bfloat16)

        for le in range(E_LOCAL):
            brows = pl.ds((my_x * E_LOCAL + le) * CAP, CAP)
            xt = jnp.dot(pmat[brows, :], x_ref[...],
                         preferred_element_type=jnp.float32
                         ).astype(jnp.bfloat16)
            ybuckets[brows, :] = ffn(xt, le)

        yr = []
        for le in range(E_LOCAL):
            xr[le].wait()
            ysend[le] = ffn(xpeer[le], le)
            r = pltpu.make_async_remote_copy(
                src_ref=ysend.at[le],
                dst_ref=ybuckets.at[pl.ds((my_x * E_LOCAL + le) * CAP, CAP), :],
                send_sem=send_sems.at[E_LOCAL + le],
                recv_sem=recv_sems.at[E_LOCAL + le],
                device_id=peer, device_id_type=pl.DeviceIdType.MESH)
            r.start()
            yr.append(r)
        for r in yr:
            r.wait()

        out_ref[...] = jnp.dot(ptmat[...], ybuckets[...],
                               preferred_element_type=jnp.float32)

    return pl.pallas_call(
        body,
        out_shape=jax.ShapeDtypeStruct((T, D), jnp.float32),
        in_specs=[pl.BlockSpec(memory_space=pltpu.VMEM)] * 5,
        out_specs=pl.BlockSpec(memory_space=pltpu.VMEM),
        scratch_shapes=[
            pltpu.VMEM((B, T), jnp.bfloat16),
            pltpu.VMEM((T, B), jnp.bfloat16),
            pltpu.VMEM((E_LOCAL, CAP, D), jnp.bfloat16),
            pltpu.VMEM((E_LOCAL, CAP, D), jnp.bfloat16),
            pltpu.VMEM((B, D), jnp.bfloat16),
            pltpu.VMEM((E_LOCAL, CAP, D), jnp.bfloat16),
            pltpu.SemaphoreType.DMA((2 * E_LOCAL,)),
            pltpu.SemaphoreType.DMA((2 * E_LOCAL,)),
        ],
        compiler_params=pltpu.CompilerParams(collective_id=0),
    )(xb, dest_row, dest_col, w1b, w2b)


def kernel(x, assign, W1, W2):
    xb = x.astype(jnp.bfloat16)
    w1b = W1.astype(jnp.bfloat16)
    w2b = W2.astype(jnp.bfloat16)

    oh = (assign[:, None] == jnp.arange(E)[None, :]).astype(jnp.int32)
    rank = jnp.cumsum(oh, axis=0) - oh
    dest = assign * CAP + jnp.sum(rank * oh, axis=1)

    return _pallas_moe(xb, dest[None, :], dest[:, None], w1b, w2b)


# device time: 61248 ns/iter; 4.1354x vs baseline; 1.0354x over previous
import jax
import jax.numpy as jnp
from jax import lax
from jax.experimental import pallas as pl
from jax.experimental.pallas import tpu as pltpu

T = 1024
D = 1024
F = 2048
E = 4
E_LOCAL = 2
CAP = 320
B = E * CAP


def _pallas_moe(xb, dest_row, dest_col, w1f, w2f):
    def body(x_ref, drow_ref, dcol_ref, w1_any, w2_any, out_ref,
             pmat, ptmat, xsend, xpeer, ybuckets, ysend,
             w1_ref, w2_ref, s1, s2,
             send_sems, recv_sems, wsems):
        my_x = lax.axis_index("x")
        my_y = lax.axis_index("y")
        my_z = lax.axis_index("z")
        q = 1 - my_x
        peer = (q, my_y, my_z)

        d10 = pltpu.make_async_copy(w1_any.at[0], s1, wsems.at[0])
        d20 = pltpu.make_async_copy(w2_any.at[0], s2, wsems.at[1])
        d10.start()
        d20.start()

        rows = lax.broadcasted_iota(jnp.int32, (B, T), 0)
        pmat[...] = (rows == drow_ref[...]).astype(jnp.bfloat16)
        cols = lax.broadcasted_iota(jnp.int32, (T, B), 1)
        ptmat[...] = (cols == dcol_ref[...]).astype(jnp.bfloat16)
        ybuckets[...] = jnp.zeros((B, D), jnp.bfloat16)

        barrier = pltpu.get_barrier_semaphore()
        pl.semaphore_signal(barrier, inc=1, device_id=peer,
                            device_id_type=pl.DeviceIdType.MESH)
        pl.semaphore_wait(barrier, 1)

        xr = []
        for j in range(E_LOCAL):
            prows = pmat[pl.ds((q * E_LOCAL + j) * CAP, CAP), :]
            xsend[j] = jnp.dot(prows, x_ref[...],
                               preferred_element_type=jnp.float32
                               ).astype(jnp.bfloat16)
            r = pltpu.make_async_remote_copy(
                src_ref=xsend.at[j], dst_ref=xpeer.at[j],
                send_sem=send_sems.at[j], recv_sem=recv_sems.at[j],
                device_id=peer, device_id_type=pl.DeviceIdType.MESH)
            r.start()
            xr.append(r)

        def ffn(xt, le):
            h = jnp.dot(xt, w1_ref[le], preferred_element_type=jnp.float32)
            h = jnp.maximum(h, 0.0).astype(jnp.bfloat16)
            y = jnp.dot(h, w2_ref[le], preferred_element_type=jnp.float32)
            return y.astype(jnp.bfloat16)

        def my_block(le):
            brows = pl.ds((my_x * E_LOCAL + le) * CAP, CAP)
            xt = jnp.dot(pmat[brows, :], x_ref[...],
                         preferred_element_type=jnp.float32
                         ).astype(jnp.bfloat16)
            ybuckets[brows, :] = ffn(xt, le)

        d10.wait()
        d20.wait()
        w1_ref[0] = s1[...].astype(jnp.bfloat16)
        w2_ref[0] = s2[...].astype(jnp.bfloat16)
        d11 = pltpu.make_async_copy(w1_any.at[1], s1, wsems.at[2])
        d21 = pltpu.make_async_copy(w2_any.at[1], s2, wsems.at[3])
        d11.start()
        d21.start()

        my_block(0)

        d11.wait()
        d21.wait()
        w1_ref[1] = s1[...].astype(jnp.bfloat16)
        w2_ref[1] = s2[...].astype(jnp.bfloat16)
        my_block(1)

        yr = []
        for le in range(E_LOCAL):
            xr[le].wait()
            ysend[le] = ffn(xpeer[le], le)
            r = pltpu.make_async_remote_copy(
                src_ref=ysend.at[le],
                dst_ref=ybuckets.at[pl.ds((my_x * E_LOCAL + le) * CAP, CAP), :],
                send_sem=send_sems.at[E_LOCAL + le],
                recv_sem=recv_sems.at[E_LOCAL + le],
                device_id=peer, device_id_type=pl.DeviceIdType.MESH)
            r.start()
            yr.append(r)
        for r in yr:
            r.wait()

        out_ref[...] = jnp.dot(ptmat[...], ybuckets[...],
                               preferred_element_type=jnp.float32)

    return pl.pallas_call(
        body,
        out_shape=jax.ShapeDtypeStruct((T, D), jnp.float32),
        in_specs=[pl.BlockSpec(memory_space=pltpu.VMEM)] * 3
        + [pl.BlockSpec(memory_space=pl.ANY)] * 2,
        out_specs=pl.BlockSpec(memory_space=pltpu.VMEM),
        scratch_shapes=[
            pltpu.VMEM((B, T), jnp.bfloat16),
            pltpu.VMEM((T, B), jnp.bfloat16),
            pltpu.VMEM((E_LOCAL, CAP, D), jnp.bfloat16),
            pltpu.VMEM((E_LOCAL, CAP, D), jnp.bfloat16),
            pltpu.VMEM((B, D), jnp.bfloat16),
            pltpu.VMEM((E_LOCAL, CAP, D), jnp.bfloat16),
            pltpu.VMEM((E_LOCAL, D, F), jnp.bfloat16),
            pltpu.VMEM((E_LOCAL, F, D), jnp.bfloat16),
            pltpu.VMEM((D, F), jnp.float32),
            pltpu.VMEM((F, D), jnp.float32),
            pltpu.SemaphoreType.DMA((2 * E_LOCAL,)),
            pltpu.SemaphoreType.DMA((2 * E_LOCAL,)),
            pltpu.SemaphoreType.DMA((4,)),
        ],
        compiler_params=pltpu.CompilerParams(
            collective_id=0, vmem_limit_bytes=60 * 1024 * 1024),
    )(xb, dest_row, dest_col, w1f, w2f)


def kernel(x, assign, W1, W2):
    xb = x.astype(jnp.bfloat16)

    oh = (assign[:, None] == jnp.arange(E)[None, :]).astype(jnp.int32)
    rank = jnp.cumsum(oh, axis=0) - oh
    dest = assign * CAP + jnp.sum(rank * oh, axis=1)

    return _pallas_moe(xb, dest[None, :], dest[:, None], W1, W2)


# device time: 53938 ns/iter; 4.6958x vs baseline; 1.1355x over previous
import jax
import jax.numpy as jnp
from jax import lax
from jax.experimental import pallas as pl
from jax.experimental.pallas import tpu as pltpu

T = 1024
D = 1024
F = 2048
E = 4
E_LOCAL = 2
CAP = 320
B = E * CAP


def _pallas_moe(xb, dest_row, dest_col, w1f, w2f):
    def body(x_ref, drow_ref, dcol_ref, w1_any, w2_any, out_ref,
             pmat, ptmat, xsend, xpeer, ybuckets, ysend,
             w1_ref, w2_ref, s1, s2,
             send_sems, recv_sems, wsems):
        my_x = lax.axis_index("x")
        my_y = lax.axis_index("y")
        my_z = lax.axis_index("z")
        q = 1 - my_x
        peer = (q, my_y, my_z)

        d10 = pltpu.make_async_copy(w1_any.at[0], s1, wsems.at[0])
        d20 = pltpu.make_async_copy(w2_any.at[0], s2, wsems.at[1])
        d10.start()
        d20.start()

        rows = lax.broadcasted_iota(jnp.int32, (B, T), 0)
        pmat[...] = (rows == drow_ref[...]).astype(jnp.bfloat16)
        cols = lax.broadcasted_iota(jnp.int32, (T, B), 1)
        ptmat[...] = (cols == dcol_ref[...]).astype(jnp.bfloat16)
        ybuckets[...] = jnp.zeros((B, D), jnp.bfloat16)

        barrier = pltpu.get_barrier_semaphore()
        pl.semaphore_signal(barrier, inc=1, device_id=peer,
                            device_id_type=pl.DeviceIdType.MESH)
        pl.semaphore_wait(barrier, 1)

        xr = []
        for j in range(E_LOCAL):
            prows = pmat[pl.ds((q * E_LOCAL + j) * CAP, CAP), :]
            xsend[j] = jnp.dot(prows, x_ref[...],
                               preferred_element_type=jnp.float32
                               ).astype(jnp.bfloat16)
            r = pltpu.make_async_remote_copy(
                src_ref=xsend.at[j], dst_ref=xpeer.at[j],
                send_sem=send_sems.at[j], recv_sem=recv_sems.at[j],
                device_id=peer, device_id_type=pl.DeviceIdType.MESH)
            r.start()
            xr.append(r)

        def ffn(xt, le):
            h = jnp.dot(xt, w1_ref[le], preferred_element_type=jnp.float32)
            h = jnp.maximum(h, 0.0).astype(jnp.bfloat16)
            y = jnp.dot(h, w2_ref[le], preferred_element_type=jnp.float32)
            return y.astype(jnp.bfloat16)

        def my_block(le):
            brows = pl.ds((my_x * E_LOCAL + le) * CAP, CAP)
            xt = jnp.dot(pmat[brows, :], x_ref[...],
                         preferred_element_type=jnp.float32
                         ).astype(jnp.bfloat16)
            ybuckets[brows, :] = ffn(xt, le)

        def y_back(le):
            xr[le].wait()
            ysend[le] = ffn(xpeer[le], le)
            r = pltpu.make_async_remote_copy(
                src_ref=ysend.at[le],
                dst_ref=ybuckets.at[pl.ds((my_x * E_LOCAL + le) * CAP, CAP), :],
                send_sem=send_sems.at[E_LOCAL + le],
                recv_sem=recv_sems.at[E_LOCAL + le],
                device_id=peer, device_id_type=pl.DeviceIdType.MESH)
            r.start()
            return r

        d10.wait()
        d20.wait()
        w1_ref[0] = s1[...].astype(jnp.bfloat16)
        w2_ref[0] = s2[...].astype(jnp.bfloat16)
        d11 = pltpu.make_async_copy(w1_any.at[1], s1, wsems.at[2])
        d21 = pltpu.make_async_copy(w2_any.at[1], s2, wsems.at[3])
        d11.start()
        d21.start()

        my_block(0)
        yr0 = y_back(0)

        d11.wait()
        d21.wait()
        w1_ref[1] = s1[...].astype(jnp.bfloat16)
        w2_ref[1] = s2[...].astype(jnp.bfloat16)
        yr1 = y_back(1)
        my_block(1)

        yr0.wait()
        yr1.wait()

        out_ref[...] = jnp.dot(ptmat[...], ybuckets[...],
                               preferred_element_type=jnp.float32)

    return pl.pallas_call(
        body,
        out_shape=jax.ShapeDtypeStruct((T, D), jnp.float32),
        in_specs=[pl.BlockSpec(memory_space=pltpu.VMEM)] * 3
        + [pl.BlockSpec(memory_space=pl.ANY)] * 2,
        out_specs=pl.BlockSpec(memory_space=pltpu.VMEM),
        scratch_shapes=[
            pltpu.VMEM((B, T), jnp.bfloat16),
            pltpu.VMEM((T, B), jnp.bfloat16),
            pltpu.VMEM((E_LOCAL, CAP, D), jnp.bfloat16),
            pltpu.VMEM((E_LOCAL, CAP, D), jnp.bfloat16),
            pltpu.VMEM((B, D), jnp.bfloat16),
            pltpu.VMEM((E_LOCAL, CAP, D), jnp.bfloat16),
            pltpu.VMEM((E_LOCAL, D, F), jnp.bfloat16),
            pltpu.VMEM((E_LOCAL, F, D), jnp.bfloat16),
            pltpu.VMEM((D, F), jnp.float32),
            pltpu.VMEM((F, D), jnp.float32),
            pltpu.SemaphoreType.DMA((2 * E_LOCAL,)),
            pltpu.SemaphoreType.DMA((2 * E_LOCAL,)),
            pltpu.SemaphoreType.DMA((4,)),
        ],
        compiler_params=pltpu.CompilerParams(
            collective_id=0, vmem_limit_bytes=60 * 1024 * 1024),
    )(xb, dest_row, dest_col, w1f, w2f)


def kernel(x, assign, W1, W2):
    xb = x.astype(jnp.bfloat16)

    oh = (assign[:, None] == jnp.arange(E)[None, :]).astype(jnp.int32)
    rank = jnp.cumsum(oh, axis=0) - oh
    dest = assign * CAP + jnp.sum(rank * oh, axis=1)

    return _pallas_moe(xb, dest[None, :], dest[:, None], W1, W2)
